# baseline (speedup 1.0000x reference)
import jax
import jax.numpy as jnp
from jax.experimental import pallas as pl
from jax.experimental.pallas import tpu as pltpu

B, N, F = 32, 27, 128
HIDDEN = 128
D_EDGE = 16
MSG = 128
PASSES = 3
OUT = 128
NP = 32
G = 8
NG = NP // G
BB = 4
RB = BB * NG * NP


def _mpnn_kernel(ez_ref, nodes_ref, Wp_ref, Op_ref, W_msg_ref, W_u_ref,
                 W_m_ref, W_gate_ref, W_gate_in_ref, W_out_ref, out_ref):
    f32 = jnp.float32
    e2 = ez_ref[...].reshape(RB, G * D_EDGE)
    nodes2 = nodes_ref[...].reshape(BB * NP, F)

    E_p = jnp.dot(e2, Wp_ref[...], preferred_element_type=f32)
    adj = jnp.dot(e2, Op_ref[...], preferred_element_type=f32)
    emask = (adj != 0.0).astype(f32)
    node_rep = jnp.sum(adj.reshape(BB * NG, NP, G * MSG), axis=1)
    node_sum2 = node_rep.reshape(BB * NP, MSG)
    nmaskF = node_sum2 != 0.0

    h2 = nodes2
    for _ in range(PASSES):
        Hm = jnp.dot(h2, W_msg_ref[...], preferred_element_type=f32)
        HmW = jnp.concatenate([Hm] * G, axis=1).reshape(BB, NP, G * MSG)
        HmR = jnp.broadcast_to(HmW[:, None, :, :], (BB, NG, NP, G * MSG))
        HmR = HmR.reshape(RB, G * MSG)
        T = jnp.tanh(E_p + HmR) * emask
        msgP = jnp.sum(T.reshape(BB * NG, NP, G * MSG), axis=1)
        msg2 = msgP.reshape(BB * NP, MSG)
        upd = jnp.tanh(
            jnp.dot(h2, W_u_ref[...], preferred_element_type=f32)
            + jnp.dot(msg2, W_m_ref[...], preferred_element_type=f32))
        h2 = jnp.where(nmaskF, upd, h2)

    gate = jax.nn.sigmoid(
        jnp.dot(h2, W_gate_ref[...], preferred_element_type=f32)
        + jnp.dot(nodes2, W_gate_in_ref[...], preferred_element_type=f32))
    emb = gate * jnp.dot(h2, W_out_ref[...], preferred_element_type=f32)
    emb = jnp.where(nmaskF, emb, 0.0).reshape(BB, NP, OUT)
    out_ref[...] = jnp.sum(emb, axis=1, keepdims=True)


@jax.jit
def kernel(nodes, edges, W_msg, W_edge, W_u, W_m, W_gate, W_gate_in, W_out):
    f32 = jnp.float32
    pad_n = NP - N
    nodes_p = jnp.pad(nodes, ((0, 0), (0, pad_n), (0, 0)))
    ep = jnp.pad(edges, ((0, 0), (0, pad_n), (0, pad_n), (0, 0)))
    ez = ep.reshape(B, NG, G, NP, D_EDGE).transpose(0, 1, 3, 2, 4)
    ez = ez.reshape(B, NG * NP, G * D_EDGE)
    eye_g = jnp.eye(G, dtype=f32)
    W_pack = (eye_g[:, None, :, None] * W_edge[None, :, None, :]
              ).reshape(G * D_EDGE, G * MSG)
    Ones_pack = (eye_g[:, None, :, None]
                 * jnp.ones((1, D_EDGE, 1, MSG), f32)
                 ).reshape(G * D_EDGE, G * MSG)

    wspec = lambda *shape: pl.BlockSpec(shape, lambda b: (0,) * len(shape))
    out = pl.pallas_call(
        _mpnn_kernel,
        grid=(B // BB,),
        in_specs=[
            pl.BlockSpec((BB, NG * NP, G * D_EDGE), lambda b: (b, 0, 0)),
            pl.BlockSpec((BB, NP, F), lambda b: (b, 0, 0)),
            wspec(G * D_EDGE, G * MSG),
            wspec(G * D_EDGE, G * MSG),
            wspec(HIDDEN, MSG),
            wspec(HIDDEN, HIDDEN),
            wspec(MSG, HIDDEN),
            wspec(HIDDEN, OUT),
            wspec(F, OUT),
            wspec(HIDDEN, OUT),
        ],
        out_specs=pl.BlockSpec((BB, 1, OUT), lambda b: (b, 0, 0)),
        out_shape=jax.ShapeDtypeStruct((B, 1, OUT), jnp.float32),
        compiler_params=pltpu.CompilerParams(
            dimension_semantics=("arbitrary",)),
    )(ez, nodes_p, W_pack, Ones_pack, W_msg, W_u, W_m, W_gate, W_gate_in,
      W_out)
    return out.reshape(B, OUT)


# speedup vs baseline: 1.4312x; 1.4312x over previous
import jax
import jax.numpy as jnp
from jax.experimental import pallas as pl
from jax.experimental.pallas import tpu as pltpu

B, N, F = 32, 27, 128
HIDDEN = 128
D_EDGE = 16
MSG = 128
PASSES = 3
OUT = 128
NP = 32
G = 8
NC = NP // G
BB = 4
ROWS = BB * NP
MASK_SHIFT = 50.0


def _mpnn_kernel(ez_ref, nodes_ref, Wp_ref, Op_ref, W_msg_ref, W_u_ref,
                 W_m_ref, W_gate_ref, W_gate_in_ref, W_out_ref, out_ref):
    f32 = jnp.float32
    e_r = ez_ref[...].reshape(ROWS, NP * D_EDGE)
    nodes2 = nodes_ref[...].reshape(ROWS, F)

    Wp = Wp_ref[...]
    Op = Op_ref[...]
    E_bm = []
    node_sum = jnp.zeros((ROWS, MSG), f32)
    n_masked = jnp.zeros((ROWS, MSG), f32)
    for c in range(NC):
        e_sl = e_r[:, c * 128:(c + 1) * 128]
        E_c = jnp.dot(e_sl, Wp, preferred_element_type=f32)
        adj_c = jnp.dot(e_sl, Op, preferred_element_type=f32)
        masked_c = (adj_c == 0.0).astype(f32)
        E_bm.append(E_c - MASK_SHIFT * masked_c)
        for g in range(G):
            sl = slice(g * MSG, (g + 1) * MSG)
            node_sum = node_sum + adj_c[:, sl]
            n_masked = n_masked + masked_c[:, sl]
    nmaskF = node_sum != 0.0

    h2 = nodes2
    for _ in range(PASSES):
        Hm = jnp.dot(h2, W_msg_ref[...], preferred_element_type=f32)
        Hm3 = Hm.reshape(BB, NP, MSG)
        msg2 = n_masked
        for c in range(NC):
            for g in range(G):
                ngh = c * G + g
                HmRep = jnp.broadcast_to(
                    Hm3[:, ngh:ngh + 1, :], (BB, NP, MSG)).reshape(ROWS, MSG)
                msg2 = msg2 + jnp.tanh(
                    E_bm[c][:, g * MSG:(g + 1) * MSG] + HmRep)
        upd = jnp.tanh(
            jnp.dot(h2, W_u_ref[...], preferred_element_type=f32)
            + jnp.dot(msg2, W_m_ref[...], preferred_element_type=f32))
        h2 = jnp.where(nmaskF, upd, h2)

    gate = jax.nn.sigmoid(
        jnp.dot(h2, W_gate_ref[...], preferred_element_type=f32)
        + jnp.dot(nodes2, W_gate_in_ref[...], preferred_element_type=f32))
    emb = gate * jnp.dot(h2, W_out_ref[...], preferred_element_type=f32)
    emb = jnp.where(nmaskF, emb, 0.0).reshape(BB, NP, OUT)
    out_ref[...] = jnp.sum(emb, axis=1, keepdims=True)


@jax.jit
def kernel(nodes, edges, W_msg, W_edge, W_u, W_m, W_gate, W_gate_in, W_out):
    f32 = jnp.float32
    pad_n = NP - N
    nodes_p = jnp.pad(nodes, ((0, 0), (0, pad_n), (0, 0)))
    ep = jnp.pad(edges, ((0, 0), (0, pad_n), (0, pad_n), (0, 0)))
    ez = ep.reshape(B, NP, NP * D_EDGE)
    eye_g = jnp.eye(G, dtype=f32)
    W_pack = (eye_g[:, None, :, None] * W_edge[None, :, None, :]
              ).reshape(G * D_EDGE, G * MSG)
    Ones_pack = (eye_g[:, None, :, None]
                 * jnp.ones((1, D_EDGE, 1, MSG), f32)
                 ).reshape(G * D_EDGE, G * MSG)

    wspec = lambda *shape: pl.BlockSpec(shape, lambda b: (0,) * len(shape))
    out = pl.pallas_call(
        _mpnn_kernel,
        grid=(B // BB,),
        in_specs=[
            pl.BlockSpec((BB, NP, NP * D_EDGE), lambda b: (b, 0, 0)),
            pl.BlockSpec((BB, NP, F), lambda b: (b, 0, 0)),
            wspec(G * D_EDGE, G * MSG),
            wspec(G * D_EDGE, G * MSG),
            wspec(HIDDEN, MSG),
            wspec(HIDDEN, HIDDEN),
            wspec(MSG, HIDDEN),
            wspec(HIDDEN, OUT),
            wspec(F, OUT),
            wspec(HIDDEN, OUT),
        ],
        out_specs=pl.BlockSpec((BB, 1, OUT), lambda b: (b, 0, 0)),
        out_shape=jax.ShapeDtypeStruct((B, 1, OUT), jnp.float32),
        compiler_params=pltpu.CompilerParams(
            dimension_semantics=("arbitrary",)),
    )(ez, nodes_p, W_pack, Ones_pack, W_msg, W_u, W_m, W_gate, W_gate_in,
      W_out)
    return out.reshape(B, OUT)


# speedup vs baseline: 1.5009x; 1.0487x over previous
import jax
import jax.numpy as jnp
from jax.experimental import pallas as pl
from jax.experimental.pallas import tpu as pltpu

B, N, F = 32, 27, 128
HIDDEN = 128
D_EDGE = 16
MSG = 128
PASSES = 3
OUT = 128
NP = 32
G = 8
NC = 3
GR = N - NC * G
BB = 4
ROWS = BB * NP
MASK_SHIFT = 50.0


def _mpnn_kernel(ez_ref, nodes_ref, Wp_ref, Op_ref, Wp_r_ref, Op_r_ref,
                 W_msg_ref, W_u_ref, W_m_ref, W_gate_ref, W_gate_in_ref,
                 W_out_ref, out_ref):
    f32 = jnp.float32
    e_r = ez_ref[...].reshape(ROWS, N * D_EDGE)
    nodes2 = nodes_ref[...].reshape(ROWS, F)

    E_bm = []
    gcounts = [G] * NC + [GR]
    node_sum = jnp.zeros((ROWS, MSG), f32)
    n_masked = jnp.zeros((ROWS, MSG), f32)
    for c in range(NC + 1):
        if c < NC:
            e_sl = e_r[:, c * 128:(c + 1) * 128]
            Wp, Op = Wp_ref[...], Op_ref[...]
        else:
            e_sl = e_r[:, NC * 128:]
            Wp, Op = Wp_r_ref[...], Op_r_ref[...]
        E_c = jnp.dot(e_sl, Wp, preferred_element_type=f32)
        adj_c = jnp.dot(e_sl, Op, preferred_element_type=f32)
        masked_c = (adj_c == 0.0).astype(f32)
        E_bm.append(E_c - MASK_SHIFT * masked_c)
        for g in range(gcounts[c]):
            sl = slice(g * MSG, (g + 1) * MSG)
            node_sum = node_sum + adj_c[:, sl]
            n_masked = n_masked + masked_c[:, sl]
    nmaskF = node_sum != 0.0

    h2 = nodes2
    for _ in range(PASSES):
        Hm = jnp.dot(h2, W_msg_ref[...], preferred_element_type=f32)
        Hm3 = Hm.reshape(BB, NP, MSG)
        msg2 = n_masked
        for c in range(NC + 1):
            for g in range(gcounts[c]):
                ngh = c * G + g
                HmRep = jnp.broadcast_to(
                    Hm3[:, ngh:ngh + 1, :], (BB, NP, MSG)).reshape(ROWS, MSG)
                msg2 = msg2 + jnp.tanh(
                    E_bm[c][:, g * MSG:(g + 1) * MSG] + HmRep)
        upd = jnp.tanh(
            jnp.dot(h2, W_u_ref[...], preferred_element_type=f32)
            + jnp.dot(msg2, W_m_ref[...], preferred_element_type=f32))
        h2 = jnp.where(nmaskF, upd, h2)

    gate = jax.nn.sigmoid(
        jnp.dot(h2, W_gate_ref[...], preferred_element_type=f32)
        + jnp.dot(nodes2, W_gate_in_ref[...], preferred_element_type=f32))
    emb = gate * jnp.dot(h2, W_out_ref[...], preferred_element_type=f32)
    emb = jnp.where(nmaskF, emb, 0.0).reshape(BB, NP, OUT)
    out_ref[...] = jnp.sum(emb, axis=1, keepdims=True)


@jax.jit
def kernel(nodes, edges, W_msg, W_edge, W_u, W_m, W_gate, W_gate_in, W_out):
    f32 = jnp.float32
    pad_n = NP - N
    nodes_p = jnp.pad(nodes, ((0, 0), (0, pad_n), (0, 0)))
    ep = jnp.pad(edges, ((0, 0), (0, pad_n), (0, 0), (0, 0)))
    ez = ep.reshape(B, NP, N * D_EDGE)
    eye_g = jnp.eye(G, dtype=f32)
    W_pack = (eye_g[:, None, :, None] * W_edge[None, :, None, :]
              ).reshape(G * D_EDGE, G * MSG)
    Ones_pack = (eye_g[:, None, :, None]
                 * jnp.ones((1, D_EDGE, 1, MSG), f32)
                 ).reshape(G * D_EDGE, G * MSG)
    eye_r = jnp.eye(GR, dtype=f32)
    W_pack_r = (eye_r[:, None, :, None] * W_edge[None, :, None, :]
                ).reshape(GR * D_EDGE, GR * MSG)
    Ones_pack_r = (eye_r[:, None, :, None]
                   * jnp.ones((1, D_EDGE, 1, MSG), f32)
                   ).reshape(GR * D_EDGE, GR * MSG)

    wspec = lambda *shape: pl.BlockSpec(shape, lambda b: (0,) * len(shape))
    out = pl.pallas_call(
        _mpnn_kernel,
        grid=(B // BB,),
        in_specs=[
            pl.BlockSpec((BB, NP, N * D_EDGE), lambda b: (b, 0, 0)),
            pl.BlockSpec((BB, NP, F), lambda b: (b, 0, 0)),
            wspec(G * D_EDGE, G * MSG),
            wspec(G * D_EDGE, G * MSG),
            wspec(GR * D_EDGE, GR * MSG),
            wspec(GR * D_EDGE, GR * MSG),
            wspec(HIDDEN, MSG),
            wspec(HIDDEN, HIDDEN),
            wspec(MSG, HIDDEN),
            wspec(HIDDEN, OUT),
            wspec(F, OUT),
            wspec(HIDDEN, OUT),
        ],
        out_specs=pl.BlockSpec((BB, 1, OUT), lambda b: (b, 0, 0)),
        out_shape=jax.ShapeDtypeStruct((B, 1, OUT), jnp.float32),
        compiler_params=pltpu.CompilerParams(
            dimension_semantics=("arbitrary",)),
    )(ez, nodes_p, W_pack, Ones_pack, W_pack_r, Ones_pack_r, W_msg, W_u,
      W_m, W_gate, W_gate_in, W_out)
    return out.reshape(B, OUT)
